# baseline (speedup 1.0000x reference)
import functools

import jax
import jax.numpy as jnp
from jax import lax
from jax.experimental import pallas as pl
from jax.experimental.pallas import tpu as pltpu
from jax.experimental.pallas import tpu_sc as plsc

LANES = 16


def _sc_pool(x, embed_table):
    B, L = x.shape
    _, D2 = embed_table.shape
    NF = D2 // LANES

    info = plsc.get_sparse_core_info()
    NC, NS = info.num_cores, info.num_subcores
    NW = NC * NS
    b_per_w = B // NW

    mesh = plsc.VectorSubcoreMesh(core_axis_name="c", subcore_axis_name="s")
    inv_l = 1.0 / L
    NBUF = 8
    n_steps = b_per_w // NBUF
    FB = 4
    UJ = 5

    @functools.partial(
        pl.kernel,
        mesh=mesh,
        out_type=jax.ShapeDtypeStruct((B, D2), jnp.float32),
        scratch_types=[
            pltpu.VMEM((2, NBUF, L), jnp.int32),
            pltpu.VMEM((NBUF, L, D2), jnp.float32),
            pltpu.VMEM((2, NBUF, D2), jnp.float32),
            pltpu.SemaphoreType.DMA,
            pltpu.SemaphoreType.DMA,
            pltpu.SemaphoreType.DMA,
            *([pltpu.SemaphoreType.DMA] * NBUF),
        ],
    )
    def k(x_hbm, table_hbm, dummy_hbm, out_hbm, idx_win, rows_v, pooled_v,
          idx_sem, fl0_sem, fl1_sem, *sems):
        fl_sems = (fl0_sem, fl1_sem)
        wid = lax.axis_index("s") * NC + lax.axis_index("c")
        base = wid * b_per_w

        pltpu.sync_copy(x_hbm.at[pl.ds(base, NBUF)], idx_win.at[0])
        pltpu.async_copy(
            x_hbm.at[pl.ds(base + NBUF, NBUF)], idx_win.at[1], idx_sem)
        for p in range(NBUF):
            pltpu.async_copy(
                table_hbm.at[idx_win.at[0, p]], rows_v.at[p], sems[p])

        def step(k2, carry):
            par = lax.rem(k2, 2)
            par_n = lax.rem(k2 + 1, 2)
            @pl.when(k2 < n_steps - 1)
            def _():
                pltpu.make_async_copy(
                    x_hbm.at[pl.ds(base, NBUF)], idx_win.at[par_n], idx_sem
                ).wait()
            @pl.when(k2 >= 2)
            def _():
                for h in range(2):
                    @pl.when(par == h)
                    def _():
                        pltpu.make_async_copy(
                            pooled_v.at[h], out_hbm.at[pl.ds(base, NBUF)],
                            fl_sems[h]).wait()

            for p in range(NBUF):
                pltpu.make_async_copy(dummy_hbm, rows_v.at[p], sems[p]).wait()
                for fg in range(0, NF, FB):
                    def acc_body(jj, accs, p=p, fg=fg):
                        jb = jj * UJ
                        for u in range(UJ):
                            accs = tuple(
                                accs[f] + rows_v[
                                    p, jb + u, pl.ds((fg + f) * LANES, LANES)]
                                for f in range(FB))
                        return accs
                    accs = lax.fori_loop(
                        0, L // UJ, acc_body,
                        tuple(jnp.zeros((LANES,), jnp.float32)
                              for _ in range(FB)))
                    for f in range(FB):
                        pooled_v[par, p, pl.ds((fg + f) * LANES, LANES)] = (
                            accs[f] * inv_l)
                @pl.when(k2 < n_steps - 1)
                def _():
                    pltpu.async_copy(
                        table_hbm.at[idx_win.at[par_n, p]],
                        rows_v.at[p], sems[p])

            @pl.when(k2 < n_steps - 2)
            def _():
                pltpu.async_copy(
                    x_hbm.at[pl.ds(base + (k2 + 2) * NBUF, NBUF)],
                    idx_win.at[par], idx_sem)
            cstart = pl.multiple_of(base + k2 * NBUF, NBUF)
            for h in range(2):
                @pl.when(par == h)
                def _():
                    pltpu.async_copy(
                        pooled_v.at[h], out_hbm.at[pl.ds(cstart, NBUF)],
                        fl_sems[h])
            return carry

        lax.fori_loop(0, n_steps, step, 0)

        for h in range(2):
            pltpu.make_async_copy(
                pooled_v.at[h], out_hbm.at[pl.ds(base, NBUF)], fl_sems[h]
            ).wait()

    return k(x, embed_table, jnp.zeros((L, D2), jnp.float32))


def _tc_head(pooled, w, bias):
    B, D2 = pooled.shape
    D = w.shape[0]
    BM = 2048

    def body(p_ref, w_ref, b_ref, o_ref):
        acc = lax.dot_general(
            p_ref[...], w_ref[...], (((1,), (1,)), ((), ())),
            preferred_element_type=jnp.float32)
        o_ref[...] = jnp.tanh(acc + b_ref[...])

    return pl.pallas_call(
        body,
        grid=(B // BM,),
        in_specs=[
            pl.BlockSpec((BM, D2), lambda i: (i, 0)),
            pl.BlockSpec((D, D2), lambda i: (0, 0)),
            pl.BlockSpec((D,), lambda i: (0,)),
        ],
        out_specs=pl.BlockSpec((BM, D), lambda i: (i, 0)),
        out_shape=jax.ShapeDtypeStruct((B, D), jnp.float32),
    )(pooled, w, bias)


def kernel(x, embed_table, W, b):
    x = x.astype(jnp.int32)
    pooled = _sc_pool(x, embed_table)
    return _tc_head(pooled, W, b)


# speedup vs baseline: 1.2025x; 1.2025x over previous
import functools

import jax
import jax.numpy as jnp
from jax import lax
from jax.experimental import pallas as pl
from jax.experimental.pallas import tpu as pltpu
from jax.experimental.pallas import tpu_sc as plsc

LANES = 16


def _sc_pool(x, embed_table):
    B, L = x.shape
    _, D2 = embed_table.shape
    NF = D2 // LANES

    info = plsc.get_sparse_core_info()
    NC, NS = info.num_cores, info.num_subcores
    NW = NC * NS
    b_per_w = B // NW

    mesh = plsc.VectorSubcoreMesh(core_axis_name="c", subcore_axis_name="s")
    inv_l = 1.0 / L
    CHUNK = 32
    NBUF = 8
    PAIR = 1
    LP = L * PAIR
    pairs_per_w = b_per_w // PAIR
    n_steps = pairs_per_w // NBUF

    @functools.partial(
        pl.kernel,
        mesh=mesh,
        out_type=jax.ShapeDtypeStruct((B, D2), jnp.float32),
        scratch_types=[
            pltpu.VMEM((pairs_per_w, LP), jnp.int32),
            pltpu.VMEM((NBUF, LP, D2), jnp.float32),
            pltpu.VMEM((CHUNK, D2), jnp.float32),
*([pltpu.SemaphoreType.DMA] * 8),
        ],
    )
    def k(x_hbm, table_hbm, dummy_hbm, out_hbm, idx_v, rows_v, pooled_v,
          *sems):
        wid = lax.axis_index("s") * NC + lax.axis_index("c")
        base = wid * b_per_w
        pbase = wid * pairs_per_w

        pltpu.sync_copy(x_hbm.at[pl.ds(pbase, pairs_per_w)], idx_v)

        for p in range(NBUF):
            pltpu.async_copy(table_hbm.at[idx_v.at[p]], rows_v.at[p], sems[p])

        rows_per_step = NBUF * PAIR
        steps_per_chunk = CHUNK // rows_per_step

        def step(i2, carry):
            for p in range(NBUF):
                pr = i2 * NBUF + p
                pltpu.make_async_copy(dummy_hbm, rows_v.at[p], sems[p]).wait()
                FB = 4
                UJ = 5
                for r in range(PAIR):
                    islot = lax.rem(pr * PAIR + r, CHUNK)
                    j0 = r * L
                    for fg in range(0, NF, FB):
                        def acc_body(jj, accs, p=p, j0=j0, fg=fg):
                            jb = j0 + jj * UJ
                            for u in range(UJ):
                                accs = tuple(
                                    accs[f] + rows_v[
                                        p, jb + u,
                                        pl.ds((fg + f) * LANES, LANES)]
                                    for f in range(FB))
                            return accs
                        accs = lax.fori_loop(
                            0, L // UJ, acc_body,
                            tuple(jnp.zeros((LANES,), jnp.float32)
                                  for _ in range(FB)))
                        for f in range(FB):
                            pooled_v[islot, pl.ds((fg + f) * LANES, LANES)] = (
                                accs[f] * inv_l)
                @pl.when(i2 < n_steps - 1)
                def _():
                    pltpu.async_copy(
                        table_hbm.at[idx_v.at[pr + NBUF]], rows_v.at[p], sems[p]
                    )
                if p == NBUF - 1:
                    @pl.when(lax.rem(i2, steps_per_chunk) == steps_per_chunk - 1)
                    def _():
                        cstart = pl.multiple_of(
                            base + (pr + 1) * PAIR - CHUNK, CHUNK)
                        pltpu.sync_copy(
                            pooled_v, out_hbm.at[pl.ds(cstart, CHUNK)]
                        )
            return carry

        lax.fori_loop(0, n_steps, step, 0)

    return k(x.reshape(B // PAIR, LP), embed_table,
             jnp.zeros((LP, D2), jnp.float32))


def _tc_head(pooled, w, bias):
    B, D2 = pooled.shape
    D = w.shape[0]
    BM = 2048

    def body(p_ref, w_ref, b_ref, o_ref):
        acc = lax.dot_general(
            p_ref[...], w_ref[...], (((1,), (1,)), ((), ())),
            preferred_element_type=jnp.float32)
        o_ref[...] = jnp.tanh(acc + b_ref[...])

    return pl.pallas_call(
        body,
        grid=(B // BM,),
        in_specs=[
            pl.BlockSpec((BM, D2), lambda i: (i, 0)),
            pl.BlockSpec((D, D2), lambda i: (0, 0)),
            pl.BlockSpec((D,), lambda i: (0,)),
        ],
        out_specs=pl.BlockSpec((BM, D), lambda i: (i, 0)),
        out_shape=jax.ShapeDtypeStruct((B, D), jnp.float32),
    )(pooled, w, bias)


def kernel(x, embed_table, W, b):
    x = x.astype(jnp.int32)
    pooled = _sc_pool(x, embed_table)
    return _tc_head(pooled, W, b)
